# baseline (speedup 1.0000x reference)
import functools

import jax
import jax.numpy as jnp
from jax import lax
from jax.experimental import pallas as pl
from jax.experimental.pallas import tpu as pltpu, tpu_sc as plsc

N = 10000
E = 320000
F_IN = 145
D = 128
G = 64

N_PAD = 10240
HALF = N_PAD // 2
SINKG = -1

NC = 2
NS = 16
CHUNK = 128
NCHUNK = 2560
CH_T = NCHUNK // NS
E_PAD = NCHUNK * CHUNK
ROWS_T = HALF // NS
ZROWS = 64

BLK = 512
NBLK = N_PAD // BLK

_HIGHEST = lax.Precision.HIGHEST


def _sc_aggr_body(h_hbm, src_hbm, dst_hbm, zeros_hbm, out_hbm,
                  src_v, dst_v, rows0, rows1, zv, aggr, sem0, sem1, ss0, ss1):
  c = lax.axis_index("c")
  s = lax.axis_index("s")
  base = c * HALF

  pltpu.sync_copy(zeros_hbm, zv)
  for k in range(ROWS_T // ZROWS):
    pltpu.sync_copy(zv.at[pl.ds(0, ZROWS)],
                    aggr.at[pl.ds(s * ROWS_T + k * ZROWS, ZROWS)])

  @pl.when(s == NS - 1)
  def _():
    pltpu.sync_copy(zv.at[pl.ds(0, NS)], aggr.at[pl.ds(HALF, NS)])

  pltpu.sync_copy(src_hbm.at[pl.ds(s * CH_T, CH_T)], src_v)
  pltpu.sync_copy(dst_hbm.at[pl.ds(s * CH_T, CH_T)], dst_v)

  def rbody(r, carry):
    for jj in range(CHUNK // 16):
      v = dst_v[r, pl.ds(jj * 16, 16)]
      loc = v - base
      ok = (v >= base) & (loc < HALF)
      dst_v[r, pl.ds(jj * 16, 16)] = jnp.where(ok, loc, HALF + s)
    return carry

  lax.fori_loop(0, CH_T, rbody, 0)
  plsc.subcore_barrier()

  pltpu.async_copy(h_hbm.at[src_v.at[0]], rows0, sem0)
  pltpu.async_copy(h_hbm.at[src_v.at[1]], rows1, sem1)

  def gbody(gi, carry):
    g = gi * 2

    pltpu.make_async_copy(h_hbm.at[src_v.at[0]], rows0, sem0).wait()
    pltpu.async_copy(rows0, aggr.at[dst_v.at[g]], ss0, add=True)

    pltpu.make_async_copy(h_hbm.at[src_v.at[1]], rows1, sem1).wait()
    pltpu.async_copy(rows1, aggr.at[dst_v.at[g + 1]], ss1, add=True)

    pltpu.make_async_copy(rows0, aggr.at[dst_v.at[0]], ss0).wait()

    @pl.when(g + 2 < CH_T)
    def _():
      pltpu.async_copy(h_hbm.at[src_v.at[g + 2]], rows0, sem0)

    pltpu.make_async_copy(rows1, aggr.at[dst_v.at[0]], ss1).wait()

    @pl.when(g + 3 < CH_T)
    def _():
      pltpu.async_copy(h_hbm.at[src_v.at[g + 3]], rows1, sem1)

    return carry

  lax.fori_loop(0, CH_T // 2, gbody, 0)
  plsc.subcore_barrier()

  pltpu.sync_copy(aggr.at[pl.ds(s * ROWS_T, ROWS_T)],
                  out_hbm.at[pl.ds(base + s * ROWS_T, ROWS_T)])


_sc_aggr = pl.kernel(
    _sc_aggr_body,
    out_type=jax.ShapeDtypeStruct((N_PAD, D), jnp.float32),
    mesh=plsc.VectorSubcoreMesh(core_axis_name="c", subcore_axis_name="s"),
    scratch_types=[
        pltpu.VMEM((CH_T, CHUNK), jnp.int32),
        pltpu.VMEM((CH_T, CHUNK), jnp.int32),
        pltpu.VMEM((CHUNK, D), jnp.float32),
        pltpu.VMEM((CHUNK, D), jnp.float32),
        pltpu.VMEM((ZROWS, D), jnp.float32),
        pltpu.VMEM_SHARED((HALF + NS, D), jnp.float32),
        pltpu.SemaphoreType.DMA,
        pltpu.SemaphoreType.DMA,
        pltpu.SemaphoreType.DMA,
        pltpu.SemaphoreType.DMA,
    ],
)


@functools.lru_cache(maxsize=None)
def _make_tc_mlp(k):
  def body(*refs):
    hs = refs[0:k]
    prs = refs[k:2 * k]
    w1s = refs[2 * k:3 * k]
    b1_ref, w2_ref, b2_ref, eps_ref = refs[3 * k:3 * k + 4]
    z_ref, stats_ref = refs[3 * k + 4:3 * k + 6]
    acc_ref = refs[3 * k + 6]
    i = pl.program_id(0)

    a = jnp.zeros((BLK, 2 * D), jnp.float32) + b1_ref[...]
    for t in range(k):
      zin = hs[t][...] * eps_ref[0, 0] + prs[t][...]
      a = a + jnp.dot(zin, w1s[t][...], preferred_element_type=jnp.float32,
                      precision=_HIGHEST)
    a = jnp.maximum(a, 0.0)
    zb = jnp.dot(a, w2_ref[...], preferred_element_type=jnp.float32,
                 precision=_HIGHEST) + b2_ref[...]
    z_ref[...] = zb

    rows = i * BLK + lax.broadcasted_iota(jnp.int32, (BLK, 1), 0)
    zm = zb * (rows < N).astype(jnp.float32)

    @pl.when(i == 0)
    def _():
      acc_ref[...] = jnp.zeros_like(acc_ref)

    acc_ref[0:1, :] += jnp.sum(zm, axis=0, keepdims=True)
    acc_ref[1:2, :] += jnp.sum(zm * zm, axis=0, keepdims=True)

    @pl.when(i == NBLK - 1)
    def _():
      mean = acc_ref[0:1, :] * (1.0 / N)
      var = acc_ref[1:2, :] * (1.0 / N) - mean * mean
      stats_ref[0:1, :] = mean
      stats_ref[1:2, :] = lax.rsqrt(var + 1e-5)

  return pl.pallas_call(
      body,
      grid=(NBLK,),
      in_specs=(
          [pl.BlockSpec((BLK, D), lambda i: (i, 0)) for _ in range(2 * k)] +
          [pl.BlockSpec((D, 2 * D), lambda i: (0, 0)) for _ in range(k)] +
          [
              pl.BlockSpec((1, 2 * D), lambda i: (0, 0)),
              pl.BlockSpec((2 * D, D), lambda i: (0, 0)),
              pl.BlockSpec((1, D), lambda i: (0, 0)),
              pl.BlockSpec((1, 1), lambda i: (0, 0)),
          ]),
      out_specs=[
          pl.BlockSpec((BLK, D), lambda i: (i, 0)),
          pl.BlockSpec((2, D), lambda i: (0, 0)),
      ],
      out_shape=[
          jax.ShapeDtypeStruct((N_PAD, D), jnp.float32),
          jax.ShapeDtypeStruct((2, D), jnp.float32),
      ],
      scratch_shapes=[pltpu.VMEM((2, D), jnp.float32)],
  )


def _norm_body(z_ref, stats_ref, g_ref, bt_ref, h_ref):
  i = pl.program_id(0)
  hn = (z_ref[...] - stats_ref[0:1, :]) * stats_ref[1:2, :] * g_ref[...]
  hn = jnp.maximum(hn + bt_ref[...], 0.0)
  rows = i * BLK + lax.broadcasted_iota(jnp.int32, (BLK, 1), 0)
  h_ref[...] = jnp.where(rows < N, hn, 0.0)


_tc_norm = pl.pallas_call(
    _norm_body,
    grid=(NBLK,),
    in_specs=[
        pl.BlockSpec((BLK, D), lambda i: (i, 0)),
        pl.BlockSpec((2, D), lambda i: (0, 0)),
        pl.BlockSpec((1, D), lambda i: (0, 0)),
        pl.BlockSpec((1, D), lambda i: (0, 0)),
    ],
    out_specs=pl.BlockSpec((BLK, D), lambda i: (i, 0)),
    out_shape=jax.ShapeDtypeStruct((N_PAD, D), jnp.float32),
)


def _pool_body(h_ref, b_ref, out_ref, acc_ref):
  i = pl.program_id(0)

  @pl.when(i == 0)
  def _():
    acc_ref[...] = jnp.zeros_like(acc_ref)

  seg = jnp.reshape(b_ref[...], (1, BLK))
  onehot = (seg == lax.broadcasted_iota(jnp.int32, (G, BLK), 0))
  acc_ref[...] += jnp.dot(onehot.astype(jnp.float32), h_ref[...],
                          preferred_element_type=jnp.float32,
                          precision=_HIGHEST)

  @pl.when(i == NBLK - 1)
  def _():
    out_ref[...] = acc_ref[...]


_tc_pool = pl.pallas_call(
    _pool_body,
    grid=(NBLK,),
    in_specs=[
        pl.BlockSpec((BLK, D), lambda i: (i, 0)),
        pl.BlockSpec((1, 1, BLK), lambda i: (i, 0, 0)),
    ],
    out_specs=pl.BlockSpec((G, D), lambda i: (0, 0)),
    out_shape=jax.ShapeDtypeStruct((G, D), jnp.float32),
    scratch_shapes=[pltpu.VMEM((G, D), jnp.float32)],
)


def kernel(x, edge_index, batch, params):
  xp = jnp.pad(x, ((0, N_PAD - N), (0, 2 * D - F_IN)))
  xa = xp[:, :D]
  xb = xp[:, D:]
  srcp = jnp.concatenate(
      [edge_index[0], jnp.zeros((E_PAD - E,), jnp.int32)]).reshape(
          NCHUNK, CHUNK)
  dstp = jnp.concatenate(
      [edge_index[1], jnp.full((E_PAD - E,), SINKG, jnp.int32)]).reshape(
          NCHUNK, CHUNK)
  batchp = jnp.concatenate(
      [batch, jnp.full((N_PAD - N,), -1, jnp.int32)]).reshape(NBLK, 1, BLK)
  zeros = jnp.zeros((ZROWS, D), jnp.float32)

  h = None
  for li, p in enumerate(params):
    b1 = p['b1'].reshape(1, 2 * D)
    b2 = p['b2'].reshape(1, D)
    eps1 = (1.0 + p['eps']).reshape(1, 1)
    if li == 0:
      w1p = jnp.pad(p['W1'], ((0, 2 * D - F_IN), (0, 0)))
      aggr_a = _sc_aggr(xa, srcp, dstp, zeros)
      aggr_b = _sc_aggr(xb, srcp, dstp, zeros)
      z, stats = _make_tc_mlp(2)(
          xa, xb, aggr_a, aggr_b, w1p[:D], w1p[D:],
          b1, p['W2'], b2, eps1)
    else:
      aggr = _sc_aggr(h, srcp, dstp, zeros)
      z, stats = _make_tc_mlp(1)(
          h, aggr, p['W1'], b1, p['W2'], b2, eps1)
    h = _tc_norm(z, stats, p['gamma'].reshape(1, D), p['beta'].reshape(1, D))

  return _tc_pool(h, batchp)


# speedup vs baseline: 1.0600x; 1.0600x over previous
import functools

import jax
import jax.numpy as jnp
from jax import lax
from jax.experimental import pallas as pl
from jax.experimental.pallas import tpu as pltpu, tpu_sc as plsc

N = 10000
E = 320000
F_IN = 145
D = 128
G = 64

N_PAD = 10240
HALF = N_PAD // 2
SINKG = -1

NC = 2
NS = 16
CHUNK = 128
NCHUNK = 2560
CH_T = NCHUNK // NS
E_PAD = NCHUNK * CHUNK
ROWS_T = HALF // NS
ZROWS = 64

BLK = 512
NBLK = N_PAD // BLK

_HIGHEST = lax.Precision.HIGHEST


def _sc_aggr_body(h_hbm, src_hbm, dst_hbm, zeros_hbm, out_hbm,
                  src_v, dst_v, rows0, rows1, zv, aggr, sem0, sem1):
  c = lax.axis_index("c")
  s = lax.axis_index("s")
  base = c * HALF

  pltpu.sync_copy(zeros_hbm, zv)
  for k in range(ROWS_T // ZROWS):
    pltpu.sync_copy(zv.at[pl.ds(0, ZROWS)],
                    aggr.at[pl.ds(s * ROWS_T + k * ZROWS, ZROWS)])

  @pl.when(s == NS - 1)
  def _():
    pltpu.sync_copy(zv.at[pl.ds(0, NS)], aggr.at[pl.ds(HALF, NS)])

  pltpu.sync_copy(src_hbm.at[pl.ds(s * CH_T, CH_T)], src_v)
  pltpu.sync_copy(dst_hbm.at[pl.ds(s * CH_T, CH_T)], dst_v)

  def rbody(r, carry):
    for jj in range(CHUNK // 16):
      v = dst_v[r, pl.ds(jj * 16, 16)]
      loc = v - base
      ok = (v >= base) & (loc < HALF)
      dst_v[r, pl.ds(jj * 16, 16)] = jnp.where(ok, loc, HALF + s)
    return carry

  lax.fori_loop(0, CH_T, rbody, 0)
  plsc.subcore_barrier()

  pltpu.async_copy(h_hbm.at[src_v.at[0]], rows0, sem0)
  pltpu.async_copy(h_hbm.at[src_v.at[1]], rows1, sem1)

  def gbody(gi, carry):
    g = gi * 2

    pltpu.make_async_copy(h_hbm.at[src_v.at[0]], rows0, sem0).wait()
    pltpu.sync_copy(rows0, aggr.at[dst_v.at[g]], add=True)

    @pl.when(g + 2 < CH_T)
    def _():
      pltpu.async_copy(h_hbm.at[src_v.at[g + 2]], rows0, sem0)

    pltpu.make_async_copy(h_hbm.at[src_v.at[1]], rows1, sem1).wait()
    pltpu.sync_copy(rows1, aggr.at[dst_v.at[g + 1]], add=True)

    @pl.when(g + 3 < CH_T)
    def _():
      pltpu.async_copy(h_hbm.at[src_v.at[g + 3]], rows1, sem1)

    return carry

  lax.fori_loop(0, CH_T // 2, gbody, 0)
  plsc.subcore_barrier()

  pltpu.sync_copy(aggr.at[pl.ds(s * ROWS_T, ROWS_T)],
                  out_hbm.at[pl.ds(base + s * ROWS_T, ROWS_T)])


_sc_aggr = pl.kernel(
    _sc_aggr_body,
    out_type=jax.ShapeDtypeStruct((N_PAD, D), jnp.float32),
    mesh=plsc.VectorSubcoreMesh(core_axis_name="c", subcore_axis_name="s"),
    scratch_types=[
        pltpu.VMEM((CH_T, CHUNK), jnp.int32),
        pltpu.VMEM((CH_T, CHUNK), jnp.int32),
        pltpu.VMEM((CHUNK, D), jnp.float32),
        pltpu.VMEM((CHUNK, D), jnp.float32),
        pltpu.VMEM((ZROWS, D), jnp.float32),
        pltpu.VMEM_SHARED((HALF + NS, D), jnp.float32),
        pltpu.SemaphoreType.DMA,
        pltpu.SemaphoreType.DMA,
    ],
)


@functools.lru_cache(maxsize=None)
def _make_tc_mlp(k):
  def body(*refs):
    hs = refs[0:k]
    prs = refs[k:2 * k]
    w1s = refs[2 * k:3 * k]
    b1_ref, w2_ref, b2_ref, eps_ref = refs[3 * k:3 * k + 4]
    z_ref, stats_ref = refs[3 * k + 4:3 * k + 6]
    acc_ref = refs[3 * k + 6]
    i = pl.program_id(0)

    a = jnp.zeros((BLK, 2 * D), jnp.float32) + b1_ref[...]
    for t in range(k):
      zin = hs[t][...] * eps_ref[0, 0] + prs[t][...]
      a = a + jnp.dot(zin, w1s[t][...], preferred_element_type=jnp.float32,
                      precision=_HIGHEST)
    a = jnp.maximum(a, 0.0)
    zb = jnp.dot(a, w2_ref[...], preferred_element_type=jnp.float32,
                 precision=_HIGHEST) + b2_ref[...]
    z_ref[...] = zb

    rows = i * BLK + lax.broadcasted_iota(jnp.int32, (BLK, 1), 0)
    zm = zb * (rows < N).astype(jnp.float32)

    @pl.when(i == 0)
    def _():
      acc_ref[...] = jnp.zeros_like(acc_ref)

    acc_ref[0:1, :] += jnp.sum(zm, axis=0, keepdims=True)
    acc_ref[1:2, :] += jnp.sum(zm * zm, axis=0, keepdims=True)

    @pl.when(i == NBLK - 1)
    def _():
      mean = acc_ref[0:1, :] * (1.0 / N)
      var = acc_ref[1:2, :] * (1.0 / N) - mean * mean
      stats_ref[0:1, :] = mean
      stats_ref[1:2, :] = lax.rsqrt(var + 1e-5)

  return pl.pallas_call(
      body,
      grid=(NBLK,),
      in_specs=(
          [pl.BlockSpec((BLK, D), lambda i: (i, 0)) for _ in range(2 * k)] +
          [pl.BlockSpec((D, 2 * D), lambda i: (0, 0)) for _ in range(k)] +
          [
              pl.BlockSpec((1, 2 * D), lambda i: (0, 0)),
              pl.BlockSpec((2 * D, D), lambda i: (0, 0)),
              pl.BlockSpec((1, D), lambda i: (0, 0)),
              pl.BlockSpec((1, 1), lambda i: (0, 0)),
          ]),
      out_specs=[
          pl.BlockSpec((BLK, D), lambda i: (i, 0)),
          pl.BlockSpec((2, D), lambda i: (0, 0)),
      ],
      out_shape=[
          jax.ShapeDtypeStruct((N_PAD, D), jnp.float32),
          jax.ShapeDtypeStruct((2, D), jnp.float32),
      ],
      scratch_shapes=[pltpu.VMEM((2, D), jnp.float32)],
  )


def _norm_body(z_ref, stats_ref, g_ref, bt_ref, h_ref):
  i = pl.program_id(0)
  hn = (z_ref[...] - stats_ref[0:1, :]) * stats_ref[1:2, :] * g_ref[...]
  hn = jnp.maximum(hn + bt_ref[...], 0.0)
  rows = i * BLK + lax.broadcasted_iota(jnp.int32, (BLK, 1), 0)
  h_ref[...] = jnp.where(rows < N, hn, 0.0)


_tc_norm = pl.pallas_call(
    _norm_body,
    grid=(NBLK,),
    in_specs=[
        pl.BlockSpec((BLK, D), lambda i: (i, 0)),
        pl.BlockSpec((2, D), lambda i: (0, 0)),
        pl.BlockSpec((1, D), lambda i: (0, 0)),
        pl.BlockSpec((1, D), lambda i: (0, 0)),
    ],
    out_specs=pl.BlockSpec((BLK, D), lambda i: (i, 0)),
    out_shape=jax.ShapeDtypeStruct((N_PAD, D), jnp.float32),
)


def _pool_body(h_ref, b_ref, out_ref, acc_ref):
  i = pl.program_id(0)

  @pl.when(i == 0)
  def _():
    acc_ref[...] = jnp.zeros_like(acc_ref)

  seg = jnp.reshape(b_ref[...], (1, BLK))
  onehot = (seg == lax.broadcasted_iota(jnp.int32, (G, BLK), 0))
  acc_ref[...] += jnp.dot(onehot.astype(jnp.float32), h_ref[...],
                          preferred_element_type=jnp.float32,
                          precision=_HIGHEST)

  @pl.when(i == NBLK - 1)
  def _():
    out_ref[...] = acc_ref[...]


_tc_pool = pl.pallas_call(
    _pool_body,
    grid=(NBLK,),
    in_specs=[
        pl.BlockSpec((BLK, D), lambda i: (i, 0)),
        pl.BlockSpec((1, 1, BLK), lambda i: (i, 0, 0)),
    ],
    out_specs=pl.BlockSpec((G, D), lambda i: (0, 0)),
    out_shape=jax.ShapeDtypeStruct((G, D), jnp.float32),
    scratch_shapes=[pltpu.VMEM((G, D), jnp.float32)],
)


def kernel(x, edge_index, batch, params):
  xp = jnp.pad(x, ((0, N_PAD - N), (0, 2 * D - F_IN)))
  xa = xp[:, :D]
  xb = xp[:, D:]
  srcp = jnp.concatenate(
      [edge_index[0], jnp.zeros((E_PAD - E,), jnp.int32)]).reshape(
          NCHUNK, CHUNK)
  dstp = jnp.concatenate(
      [edge_index[1], jnp.full((E_PAD - E,), SINKG, jnp.int32)]).reshape(
          NCHUNK, CHUNK)
  batchp = jnp.concatenate(
      [batch, jnp.full((N_PAD - N,), -1, jnp.int32)]).reshape(NBLK, 1, BLK)
  zeros = jnp.zeros((ZROWS, D), jnp.float32)

  h = None
  for li, p in enumerate(params):
    b1 = p['b1'].reshape(1, 2 * D)
    b2 = p['b2'].reshape(1, D)
    eps1 = (1.0 + p['eps']).reshape(1, 1)
    if li == 0:
      w1p = jnp.pad(p['W1'], ((0, 2 * D - F_IN), (0, 0)))
      aggr_a = _sc_aggr(xa, srcp, dstp, zeros)
      aggr_b = _sc_aggr(xb, srcp, dstp, zeros)
      z, stats = _make_tc_mlp(2)(
          xa, xb, aggr_a, aggr_b, w1p[:D], w1p[D:],
          b1, p['W2'], b2, eps1)
    else:
      aggr = _sc_aggr(h, srcp, dstp, zeros)
      z, stats = _make_tc_mlp(1)(
          h, aggr, p['W1'], b1, p['W2'], b2, eps1)
    h = _tc_norm(z, stats, p['gamma'].reshape(1, D), p['beta'].reshape(1, D))

  return _tc_pool(h, batchp)
